# Initial kernel scaffold; baseline (speedup 1.0000x reference)
import jax
import jax.numpy as jnp
from jax.experimental import pallas as pl


def kernel(x, w, gamma, beta, alpha):
    raise NotImplementedError("write your pallas kernel here")


# speedup vs baseline: 1.7217x; 1.7217x over previous
import functools

import jax
import jax.numpy as jnp
from jax.experimental import pallas as pl
from jax.experimental.pallas import tpu as pltpu


def _conv_stats_kernel(x_ref, w_ref, y_ref, s_ref, *, H, W, B, Cin):
    hw = H * W
    xb = x_ref[...].reshape(B * Cin, hw).astype(jnp.bfloat16)

    lane = jax.lax.broadcasted_iota(jnp.int32, (1, hw), 1)
    h_idx = lane // W
    w_idx = lane % W

    taps = []
    for dh in (-1, 0, 1):
        for dw in (-1, 0, 1):
            off = dh * W + dw
            shifted = xb if off == 0 else pltpu.roll(xb, (-off) % hw, axis=1)
            valid = ((h_idx + dh >= 0) & (h_idx + dh < H) &
                     (w_idx + dw >= 0) & (w_idx + dw < W))
            taps.append(jnp.where(valid, shifted, jnp.bfloat16(0)))

    cout = w_ref.shape[0]
    ssum = jnp.zeros((cout, 1), jnp.float32)
    ssq = jnp.zeros((cout, 1), jnp.float32)
    for b in range(B):
        x9 = jnp.concatenate([t[b * Cin:(b + 1) * Cin] for t in taps], axis=0)
        acc = jnp.dot(w_ref[...], x9, preferred_element_type=jnp.float32)
        y_ref[b] = acc.astype(jnp.bfloat16)
        ssum = ssum + jnp.sum(acc, axis=1, keepdims=True)
        ssq = ssq + jnp.sum(acc * acc, axis=1, keepdims=True)
    s_ref[0] = jnp.concatenate([ssum, ssq], axis=1)


def _bn_prelu_kernel(y_ref, p_ref, o_ref):
    y = y_ref[...].astype(jnp.float32)
    scale = p_ref[:, 0:1]
    shift = p_ref[:, 1:2]
    alpha = p_ref[:, 2:3]
    z = y * scale + shift
    o_ref[...] = jnp.where(z >= 0, z, alpha * z)


def kernel(x, w, gamma, beta, alpha, *, eps=1e-5):
    N, Cin, H, W = x.shape
    Cout, Cin_w, KH, KW = w.shape
    assert (KH, KW) == (3, 3) and Cin_w == Cin
    HW = H * W
    M = N * HW

    B = 4 if N % 4 == 0 else 1
    B2 = 8 if N % 8 == 0 else 1

    x_r = x.reshape(N, Cin, HW)
    w_cat = jnp.transpose(w, (0, 2, 3, 1)).reshape(Cout, 9 * Cin)
    w_cat = w_cat.astype(jnp.bfloat16)

    y_t, stats = pl.pallas_call(
        functools.partial(_conv_stats_kernel, H=H, W=W, B=B, Cin=Cin),
        out_shape=(jax.ShapeDtypeStruct((N, Cout, HW), jnp.bfloat16),
                   jax.ShapeDtypeStruct((N // B, Cout, 2), jnp.float32)),
        grid=(N // B,),
        in_specs=[
            pl.BlockSpec((B, Cin, HW), lambda n: (n, 0, 0)),
            pl.BlockSpec((Cout, 9 * Cin), lambda n: (0, 0)),
        ],
        out_specs=(
            pl.BlockSpec((B, Cout, HW), lambda n: (n, 0, 0)),
            pl.BlockSpec((1, Cout, 2), lambda n: (n, 0, 0)),
        ),
        compiler_params=pltpu.CompilerParams(
            dimension_semantics=("parallel",)),
    )(x_r, w_cat)

    s = jnp.sum(stats, axis=0)
    mean = s[:, 0] / M
    var = s[:, 1] / M - mean * mean
    inv_std = jax.lax.rsqrt(var + eps)
    scale = gamma.astype(jnp.float32) * inv_std
    shift = beta.astype(jnp.float32) - mean * scale
    params = jnp.stack([scale, shift, alpha.astype(jnp.float32)], axis=1)

    out_t = pl.pallas_call(
        _bn_prelu_kernel,
        out_shape=jax.ShapeDtypeStruct((N, Cout, HW), jnp.float32),
        grid=(N // B2,),
        in_specs=[
            pl.BlockSpec((B2, Cout, HW), lambda n: (n, 0, 0)),
            pl.BlockSpec((Cout, 3), lambda n: (0, 0)),
        ],
        out_specs=pl.BlockSpec((B2, Cout, HW), lambda n: (n, 0, 0)),
        compiler_params=pltpu.CompilerParams(
            dimension_semantics=("parallel",)),
    )(y_t, params)

    return out_t.reshape(N, Cout, H, W)
